# baseline (speedup 1.0000x reference)
import jax
import jax.numpy as jnp
import numpy as np
from jax import lax
from jax.experimental import pallas as pl
from jax.experimental.pallas import tpu as pltpu
from jax.experimental.pallas import tpu_sc as plsc

R = 38226
DEG = 16
D = 128
B = 277
L = B - 1
HOPS = 4
VOCAB1 = 1001
VPAD = 1008

NW = 32
RPAD = 38400

EC = 120
ENB = RPAD // EC
EPW = ENB // NW


FC = 92
GPW = 9

ZROW = R

TILE = 768

_mesh = plsc.VectorSubcoreMesh(core_axis_name="c", subcore_axis_name="s")


def _wid():
    return lax.axis_index("s") * 2 + lax.axis_index("c")


def _tables_body(emb, w0, w1, w2, w3, o0, o1, o2, o3):
    e = emb[...]
    o0[...] = jnp.dot(e, w0[...], preferred_element_type=jnp.float32)
    o1[...] = jnp.dot(e, w1[...], preferred_element_type=jnp.float32)
    o2[...] = jnp.dot(e, w2[...], preferred_element_type=jnp.float32)
    o3[...] = jnp.dot(e, w3[...], preferred_element_type=jnp.float32)


_tables_call = pl.pallas_call(
    _tables_body,
    out_shape=[jax.ShapeDtypeStruct((VPAD, D), jnp.float32)] * 4,
)


def _embed_body(feat, t0, t1, t2, t3, o0, o1, o2, o3, idx_v, r0, r1, r2, sem,
                osem):
    w = _wid()
    tabs = (t0, t1, t2, t3)
    outs = (o0, o1, o2, o3)
    rv = (r0, r1, r2)
    pltpu.sync_copy(feat.at[w], idx_v)

    def start_in(u):
        c, t = divmod(u, 4)
        return pltpu.async_copy(tabs[t].at[idx_v.at[c]], rv[u % 3], sem)

    U = 4 * EPW
    in_cp = {0: start_in(0), 1: start_in(1), 2: start_in(2)}
    out_cp = {}
    for u in range(U):
        c, t = divmod(u, 4)
        in_cp[u % 3].wait()
        out_cp[u % 3] = pltpu.async_copy(
            rv[u % 3], outs[t].at[pl.ds((w * EPW + c) * EC, EC)], osem
        )
        if u + 3 < U:
            out_cp[u % 3].wait()
            in_cp[u % 3] = start_in(u + 3)
    for s_ in range(3):
        if s_ in out_cp:
            out_cp[s_].wait()


_embed_call = pl.kernel(
    _embed_body,
    mesh=_mesh,
    out_type=[jax.ShapeDtypeStruct((RPAD, D), jnp.float32)] * 4,
    scratch_types=[
        pltpu.VMEM((EPW, EC), jnp.int32),
        pltpu.VMEM((EC, D), jnp.float32),
        pltpu.VMEM((EC, D), jnp.float32),
        pltpu.VMEM((EC, D), jnp.float32),
        pltpu.SemaphoreType.DMA,
        pltpu.SemaphoreType.DMA,
    ],
)


MG = 15
MBPW = 1200 // (8 * MG)
MB_C0 = 13


def _mean_body(y, adjf, m, idx_v, r0, r1, r2, out_v, sem):
    c = lax.axis_index("c")
    s = lax.axis_index("s")
    start = s * (2 * MBPW) + jnp.where(c == 0, 0, MB_C0)
    cnt = jnp.where(c == 0, MB_C0, 2 * MBPW - MB_C0)
    rv = (r0, r1, r2)

    def block(b, carry):
        blk = start + b
        gbase = blk * MG
        pltpu.sync_copy(adjf.at[blk], idx_v)
        cps = {
            0: pltpu.async_copy(y.at[idx_v.at[0]], r0, sem),
            1: pltpu.async_copy(y.at[idx_v.at[1]], r1, sem),
        }
        for gi in range(MG):
            if gi + 2 < MG:
                cps[(gi + 2) % 3] = pltpu.async_copy(
                    y.at[idx_v.at[gi + 2]], rv[(gi + 2) % 3], sem
                )
            cps[gi % 3].wait()
            rows = rv[gi % 3]

            def node(n, c2):
                rid = (gbase + gi) * 8 + n
                scale = jnp.where(rid < R, 1.0 / DEG, 0.0)
                for c in range(D // 16):
                    vs = [
                        rows[n * DEG + j, pl.ds(c * 16, 16)]
                        for j in range(DEG)
                    ]
                    while len(vs) > 1:
                        vs = [vs[i] + vs[i + 1] for i in range(0, len(vs), 2)]
                    out_v[gi * 8 + n, pl.ds(c * 16, 16)] = vs[0] * scale
                return c2

            lax.fori_loop(0, 8, node, 0)
        pltpu.sync_copy(out_v, m.at[pl.ds(gbase * 8, 8 * MG)])
        return carry

    lax.fori_loop(0, cnt, block, 0)


_mean_call = pl.kernel(
    _mean_body,
    mesh=_mesh,
    out_type=jax.ShapeDtypeStruct((RPAD, D), jnp.float32),
    scratch_types=[
        pltpu.VMEM((MG, 128), jnp.int32),
        pltpu.VMEM((128, D), jnp.float32),
        pltpu.VMEM((128, D), jnp.float32),
        pltpu.VMEM((128, D), jnp.float32),
        pltpu.VMEM((8 * MG, D), jnp.float32),
        pltpu.SemaphoreType.DMA,
    ],
)


def _hop_body(spf, mpf, spb, mpb, wsf, wnf, wsb, wnb, osf, oyf, osb, oyb):
    rid = lax.broadcasted_iota(jnp.int32, (TILE, 1), 0) + pl.program_id(0) * TILE
    mask = rid < R
    for sp, mp, ws, wn, os_, oy in (
        (spf, mpf, wsf, wnf, osf, oyf),
        (spb, mpb, wsb, wnb, osb, oyb),
    ):
        a = jnp.maximum(sp[...], 0.0)
        bb = jnp.maximum(mp[...], 0.0)
        w = ws[...]
        s_out = jnp.dot(a, w[:D], preferred_element_type=jnp.float32) + jnp.dot(
            bb, w[D:], preferred_element_type=jnp.float32
        )
        os_[...] = jnp.where(mask, s_out, 0.0)
        w = wn[...]
        y_out = jnp.dot(a, w[:D], preferred_element_type=jnp.float32) + jnp.dot(
            bb, w[D:], preferred_element_type=jnp.float32
        )
        oy[...] = jnp.where(mask, y_out, 0.0)


_hop_call = pl.pallas_call(
    _hop_body,
    grid=(RPAD // TILE,),
    in_specs=[pl.BlockSpec((TILE, D), lambda i: (i, 0))] * 4
    + [pl.BlockSpec((2 * D, D), lambda i: (0, 0))] * 4,
    out_specs=[pl.BlockSpec((TILE, D), lambda i: (i, 0))] * 4,
    out_shape=[jax.ShapeDtypeStruct((RPAD, D), jnp.float32)] * 4,
)


FB = 288


GPP = 18
G_C0 = 12


def _final_body(p0, p1, p2, p3, meta, hid, pool, meta_v, b0, b1,
                pool_v, sem, osem):
    c = lax.axis_index("c")
    sub = lax.axis_index("s")
    pltpu.sync_copy(meta, meta_v)
    parts = (p0, p1, p2, p3)
    bufs = (b0, b1)

    def pipeline(glist):
        units = [(g, k) for g in glist for k in range(4)]
        U = len(units)

        def start_in(u):
            g, k = units[u]
            s = meta_v[g, :][0]
            s8 = pl.multiple_of(s & ~7, 8)
            return pltpu.async_copy(
                parts[k].at[pl.ds(s8, FB)], bufs[u % 2], sem)

        in_cp = {0: start_in(0), 1: start_in(1)}
        out_cp = {}
        for u in range(U):
            g, k = units[u]
            buf = bufs[u % 2]
            mrow = meta_v[g, :]
            s = mrow[0]
            n = mrow[1]
            off = s - (s & ~7)
            in_cp[u % 2].wait()

            def row(r, cc):
                pred = r < n
                out = []
                for v in range(D // 16):
                    x = jnp.maximum(buf[r + off, pl.ds(v * 16, 16)], 0.0)
                    x = jnp.where(pred, x, 0.0)
                    buf[r + off, pl.ds(v * 16, 16)] = x
                    out.append(jnp.maximum(cc[v], x))
                return tuple(out)

            car = tuple(
                jnp.zeros((16,), jnp.float32) for _ in range(D // 16))
            car = lax.fori_loop(0, L, row, car)
            for v in range(D // 16):
                pool_v[pl.ds(k * D + v * 16, 16)] = car[v]
            out_cp[u % 2] = pltpu.async_copy(
                buf.at[pl.ds(off, L)], hid.at[g, :, pl.ds(k * D, D)], osem
            )
            if u + 2 < U:
                out_cp[u % 2].wait()
                in_cp[u % 2] = start_in(u + 2)
            if k == 3:
                pltpu.sync_copy(pool_v, pool.at[g])
        for cp in out_cp.values():
            cp.wait()

    @pl.when(c == 0)
    def _():
        pipeline([jnp.minimum(sub * GPP + i, B - 1) for i in range(G_C0)])

    @pl.when(c == 1)
    def _():
        pipeline([
            jnp.minimum(sub * GPP + G_C0 + i, B - 1)
            for i in range(GPP - G_C0)
        ])


_final_call = pl.kernel(
    _final_body,
    mesh=_mesh,
    out_type=[
        jax.ShapeDtypeStruct((B, L, 4 * D), jnp.float32),
        jax.ShapeDtypeStruct((B, 4 * D), jnp.float32),
    ],
    scratch_types=[
        pltpu.VMEM((288, 16), jnp.int32),
        pltpu.VMEM((FB, D), jnp.float32),
        pltpu.VMEM((FB, D), jnp.float32),
        pltpu.VMEM((4 * D,), jnp.float32),
        pltpu.SemaphoreType.DMA,
        pltpu.SemaphoreType.DMA,
    ],
)


def _chunk_adj(adj):
    a = jnp.zeros((RPAD, DEG), jnp.int32).at[:R].set(adj.astype(jnp.int32))
    return a.reshape(RPAD * DEG // (128 * MG), MG, 128)


def kernel(fw_adjs, bw_adjs, features, num_nodes, emb, Wsf0, Wnf0, Wsf, Wnf,
           Wsb0, Wnb0, Wsb, Wnb):
    featp = jnp.zeros((RPAD,), jnp.int32).at[:R].set(features.astype(jnp.int32))
    featp = featp.reshape(NW, EPW, EC)
    fadj = _chunk_adj(fw_adjs)
    badj = _chunk_adj(bw_adjs)
    embp = jnp.zeros((VPAD, D), jnp.float32).at[:VOCAB1].set(emb)

    tsf, tyf, tsb, tyb = _tables_call(embp, Wsf0, Wnf0, Wsb0, Wnb0)
    sf, yf, sb, yb = _embed_call(featp, tsf, tyf, tsb, tyb)
    mf = _mean_call(yf, fadj)
    mb = _mean_call(yb, badj)
    for h in range(1, HOPS):
        sf, yf, sb, yb = _hop_call(sf, mf, sb, mb, Wsf[h - 1], Wnf[h - 1],
                                   Wsb[h - 1], Wnb[h - 1])
        mf = _mean_call(yf, fadj)
        mb = _mean_call(yb, badj)

    nn = num_nodes.astype(jnp.int32)
    csum = jnp.cumsum(nn)
    starts = csum - nn
    meta = jnp.zeros((288, 16), jnp.int32)
    meta = meta.at[:B, 0].set(starts).at[:B, 1].set(nn)

    hidden, pooled = _final_call(sf, mf, sb, mb, meta)
    return hidden, pooled, pooled


# speedup vs baseline: 1.0065x; 1.0065x over previous
import jax
import jax.numpy as jnp
import numpy as np
from jax import lax
from jax.experimental import pallas as pl
from jax.experimental.pallas import tpu as pltpu
from jax.experimental.pallas import tpu_sc as plsc

R = 38226
DEG = 16
D = 128
B = 277
L = B - 1
HOPS = 4
VOCAB1 = 1001
VPAD = 1008

NW = 32
RPAD = 38400

EC = 120
ENB = RPAD // EC
EPW = ENB // NW


FC = 92
GPW = 9

ZROW = R

TILE = 768

_mesh = plsc.VectorSubcoreMesh(core_axis_name="c", subcore_axis_name="s")


def _wid():
    return lax.axis_index("s") * 2 + lax.axis_index("c")


def _tables_body(emb, w0, w1, w2, w3, o0, o1, o2, o3):
    e = emb[...]
    o0[...] = jnp.dot(e, w0[...], preferred_element_type=jnp.float32)
    o1[...] = jnp.dot(e, w1[...], preferred_element_type=jnp.float32)
    o2[...] = jnp.dot(e, w2[...], preferred_element_type=jnp.float32)
    o3[...] = jnp.dot(e, w3[...], preferred_element_type=jnp.float32)


_tables_call = pl.pallas_call(
    _tables_body,
    out_shape=[jax.ShapeDtypeStruct((VPAD, D), jnp.float32)] * 4,
)


def _embed_body(feat, t0, t1, t2, t3, o0, o1, o2, o3, idx_v, r0, r1, r2, sem,
                osem):
    w = _wid()
    tabs = (t0, t1, t2, t3)
    outs = (o0, o1, o2, o3)
    rv = (r0, r1, r2)
    pltpu.sync_copy(feat.at[w], idx_v)

    def start_in(u):
        c, t = divmod(u, 4)
        return pltpu.async_copy(tabs[t].at[idx_v.at[c]], rv[u % 3], sem)

    U = 4 * EPW
    in_cp = {0: start_in(0), 1: start_in(1), 2: start_in(2)}
    out_cp = {}
    for u in range(U):
        c, t = divmod(u, 4)
        in_cp[u % 3].wait()
        out_cp[u % 3] = pltpu.async_copy(
            rv[u % 3], outs[t].at[pl.ds((w * EPW + c) * EC, EC)], osem
        )
        if u + 3 < U:
            out_cp[u % 3].wait()
            in_cp[u % 3] = start_in(u + 3)
    for s_ in range(3):
        if s_ in out_cp:
            out_cp[s_].wait()


_embed_call = pl.kernel(
    _embed_body,
    mesh=_mesh,
    out_type=[jax.ShapeDtypeStruct((RPAD, D), jnp.float32)] * 4,
    scratch_types=[
        pltpu.VMEM((EPW, EC), jnp.int32),
        pltpu.VMEM((EC, D), jnp.float32),
        pltpu.VMEM((EC, D), jnp.float32),
        pltpu.VMEM((EC, D), jnp.float32),
        pltpu.SemaphoreType.DMA,
        pltpu.SemaphoreType.DMA,
    ],
)


MG = 15
MBPW = 1200 // (8 * MG)
MB_C0 = 13


def _mean_body(y, adjf, m, idx_v, r0, r1, r2, out_v, sem):
    c = lax.axis_index("c")
    s = lax.axis_index("s")
    start = s * (2 * MBPW) + jnp.where(c == 0, 0, MB_C0)
    cnt = jnp.where(c == 0, MB_C0, 2 * MBPW - MB_C0)
    rv = (r0, r1, r2)

    def block(b, carry):
        blk = start + b
        gbase = blk * MG
        pltpu.sync_copy(adjf.at[blk], idx_v)
        cps = {
            0: pltpu.async_copy(y.at[idx_v.at[0]], r0, sem),
            1: pltpu.async_copy(y.at[idx_v.at[1]], r1, sem),
        }
        for gi in range(MG):
            if gi + 2 < MG:
                cps[(gi + 2) % 3] = pltpu.async_copy(
                    y.at[idx_v.at[gi + 2]], rv[(gi + 2) % 3], sem
                )
            cps[gi % 3].wait()
            rows = rv[gi % 3]

            def node(n, c2):
                rid = (gbase + gi) * 8 + n
                scale = jnp.where(rid < R, 1.0 / DEG, 0.0)
                for c in range(D // 16):
                    vs = [
                        rows[n * DEG + j, pl.ds(c * 16, 16)]
                        for j in range(DEG)
                    ]
                    while len(vs) > 1:
                        vs = [vs[i] + vs[i + 1] for i in range(0, len(vs), 2)]
                    out_v[gi * 8 + n, pl.ds(c * 16, 16)] = vs[0] * scale
                return c2

            lax.fori_loop(0, 8, node, 0)
        pltpu.sync_copy(out_v, m.at[pl.ds(gbase * 8, 8 * MG)])
        return carry

    lax.fori_loop(0, cnt, block, 0)


_mean_call = pl.kernel(
    _mean_body,
    mesh=_mesh,
    out_type=jax.ShapeDtypeStruct((RPAD, D), jnp.float32),
    scratch_types=[
        pltpu.VMEM((MG, 128), jnp.int32),
        pltpu.VMEM((128, D), jnp.float32),
        pltpu.VMEM((128, D), jnp.float32),
        pltpu.VMEM((128, D), jnp.float32),
        pltpu.VMEM((8 * MG, D), jnp.float32),
        pltpu.SemaphoreType.DMA,
    ],
)


def _hop_body(spf, mpf, spb, mpb, wsf, wnf, wsb, wnb, osf, oyf, osb, oyb):
    rid = lax.broadcasted_iota(jnp.int32, (TILE, 1), 0) + pl.program_id(0) * TILE
    mask = rid < R
    for sp, mp, ws, wn, os_, oy in (
        (spf, mpf, wsf, wnf, osf, oyf),
        (spb, mpb, wsb, wnb, osb, oyb),
    ):
        a = jnp.maximum(sp[...], 0.0)
        bb = jnp.maximum(mp[...], 0.0)
        w = ws[...]
        s_out = jnp.dot(a, w[:D], preferred_element_type=jnp.float32) + jnp.dot(
            bb, w[D:], preferred_element_type=jnp.float32
        )
        os_[...] = jnp.where(mask, s_out, 0.0)
        w = wn[...]
        y_out = jnp.dot(a, w[:D], preferred_element_type=jnp.float32) + jnp.dot(
            bb, w[D:], preferred_element_type=jnp.float32
        )
        oy[...] = jnp.where(mask, y_out, 0.0)


_hop_call = pl.pallas_call(
    _hop_body,
    grid=(RPAD // TILE,),
    in_specs=[pl.BlockSpec((TILE, D), lambda i: (i, 0))] * 4
    + [pl.BlockSpec((2 * D, D), lambda i: (0, 0))] * 4,
    out_specs=[pl.BlockSpec((TILE, D), lambda i: (i, 0))] * 4,
    out_shape=[jax.ShapeDtypeStruct((RPAD, D), jnp.float32)] * 4,
)


FB = 288


def _final_body(p0, p1, p2, p3, meta, hid, pool, meta_v, b0, b1,
                pool_v, sem, osem):
    w = _wid()
    pltpu.sync_copy(meta, meta_v)
    parts = (p0, p1, p2, p3)
    bufs = (b0, b1)
    units = []
    for gi in range(GPW):
        g = jnp.minimum(w + NW * gi, B - 1)
        for k in range(4):
            units.append((g, k))
    U = len(units)

    def start_in(u):
        g, k = units[u]
        s = meta_v[g, :][0]
        s8 = pl.multiple_of(s & ~7, 8)
        return pltpu.async_copy(parts[k].at[pl.ds(s8, FB)], bufs[u % 2], sem)

    in_cp = {0: start_in(0), 1: start_in(1)}
    out_cp = {}

    for u in range(U):
        g, k = units[u]
        buf = bufs[u % 2]
        mrow = meta_v[g, :]
        s = mrow[0]
        n = mrow[1]
        off = s - (s & ~7)
        in_cp[u % 2].wait()

        def row(r, cc):
            pred = r < n
            out = []
            for v in range(D // 16):
                x = jnp.maximum(buf[r + off, pl.ds(v * 16, 16)], 0.0)
                x = jnp.where(pred, x, 0.0)
                buf[r + off, pl.ds(v * 16, 16)] = x
                out.append(jnp.maximum(cc[v], x))
            return tuple(out)

        car = tuple(jnp.zeros((16,), jnp.float32) for _ in range(D // 16))
        car = lax.fori_loop(0, L, row, car)
        for v in range(D // 16):
            pool_v[pl.ds(k * D + v * 16, 16)] = car[v]
        out_cp[u % 2] = pltpu.async_copy(
            buf.at[pl.ds(off, L)], hid.at[g, :, pl.ds(k * D, D)], osem
        )
        if u + 2 < U:
            out_cp[u % 2].wait()
            in_cp[u % 2] = start_in(u + 2)
        if k == 3:
            pltpu.sync_copy(pool_v, pool.at[g])
    for cp in out_cp.values():
        cp.wait()


_final_call = pl.kernel(
    _final_body,
    mesh=_mesh,
    out_type=[
        jax.ShapeDtypeStruct((B, L, 4 * D), jnp.float32),
        jax.ShapeDtypeStruct((B, 4 * D), jnp.float32),
    ],
    scratch_types=[
        pltpu.VMEM((288, 16), jnp.int32),
        pltpu.VMEM((FB, D), jnp.float32),
        pltpu.VMEM((FB, D), jnp.float32),
        pltpu.VMEM((4 * D,), jnp.float32),
        pltpu.SemaphoreType.DMA,
        pltpu.SemaphoreType.DMA,
    ],
)


def _chunk_adj(adj):
    a = jnp.zeros((RPAD, DEG), jnp.int32).at[:R].set(adj.astype(jnp.int32))
    return a.reshape(RPAD * DEG // (128 * MG), MG, 128)


def kernel(fw_adjs, bw_adjs, features, num_nodes, emb, Wsf0, Wnf0, Wsf, Wnf,
           Wsb0, Wnb0, Wsb, Wnb):
    featp = jnp.zeros((RPAD,), jnp.int32).at[:R].set(features.astype(jnp.int32))
    featp = featp.reshape(NW, EPW, EC)
    fadj = _chunk_adj(fw_adjs)
    badj = _chunk_adj(bw_adjs)
    embp = jnp.zeros((VPAD, D), jnp.float32).at[:VOCAB1].set(emb)

    tsf, tyf, tsb, tyb = _tables_call(embp, Wsf0, Wnf0, Wsb0, Wnb0)
    sf, yf, sb, yb = _embed_call(featp, tsf, tyf, tsb, tyb)
    mf = _mean_call(yf, fadj)
    mb = _mean_call(yb, badj)
    for h in range(1, HOPS):
        sf, yf, sb, yb = _hop_call(sf, mf, sb, mb, Wsf[h - 1], Wnf[h - 1],
                                   Wsb[h - 1], Wnb[h - 1])
        mf = _mean_call(yf, fadj)
        mb = _mean_call(yb, badj)

    nn = num_nodes.astype(jnp.int32)
    csum = jnp.cumsum(nn)
    starts = csum - nn
    meta = jnp.zeros((288, 16), jnp.int32)
    meta = meta.at[:B, 0].set(starts).at[:B, 1].set(nn)

    hidden, pooled = _final_call(sf, mf, sb, mb, meta)
    return hidden, pooled, pooled


# speedup vs baseline: 1.0071x; 1.0006x over previous
import jax
import jax.numpy as jnp
import numpy as np
from jax import lax
from jax.experimental import pallas as pl
from jax.experimental.pallas import tpu as pltpu
from jax.experimental.pallas import tpu_sc as plsc

R = 38226
DEG = 16
D = 128
B = 277
L = B - 1
HOPS = 4
VOCAB1 = 1001
VPAD = 1008

NW = 32
RPAD = 38400

EC = 120
ENB = RPAD // EC
EPW = ENB // NW


FC = 92
GPW = 9

ZROW = R

TILE = 768

_mesh = plsc.VectorSubcoreMesh(core_axis_name="c", subcore_axis_name="s")


def _wid():
    return lax.axis_index("s") * 2 + lax.axis_index("c")


def _tables_body(emb, w0, w1, w2, w3, o0, o1, o2, o3):
    e = emb[...]
    o0[...] = jnp.dot(e, w0[...], preferred_element_type=jnp.float32)
    o1[...] = jnp.dot(e, w1[...], preferred_element_type=jnp.float32)
    o2[...] = jnp.dot(e, w2[...], preferred_element_type=jnp.float32)
    o3[...] = jnp.dot(e, w3[...], preferred_element_type=jnp.float32)


_tables_call = pl.pallas_call(
    _tables_body,
    out_shape=[jax.ShapeDtypeStruct((VPAD, D), jnp.float32)] * 4,
)


def _embed_body(feat, t0, t1, t2, t3, o0, o1, o2, o3, idx_v, r0, r1, r2, r3,
                r4, sem, osem):
    w = _wid()
    tabs = (t0, t1, t2, t3)
    outs = (o0, o1, o2, o3)
    rv = (r0, r1, r2, r3, r4)
    pltpu.sync_copy(feat.at[w], idx_v)

    NB = 5

    def start_in(u):
        c, t = divmod(u, 4)
        return pltpu.async_copy(tabs[t].at[idx_v.at[c]], rv[u % NB], sem)

    U = 4 * EPW
    in_cp = {u: start_in(u) for u in range(NB)}
    out_cp = {}
    for u in range(U):
        c, t = divmod(u, 4)
        in_cp[u % NB].wait()
        out_cp[u % NB] = pltpu.async_copy(
            rv[u % NB], outs[t].at[pl.ds((w * EPW + c) * EC, EC)], osem
        )
        if u + NB < U:
            out_cp[u % NB].wait()
            in_cp[u % NB] = start_in(u + NB)
    for s_ in range(NB):
        if s_ in out_cp:
            out_cp[s_].wait()


_embed_call = pl.kernel(
    _embed_body,
    mesh=_mesh,
    out_type=[jax.ShapeDtypeStruct((RPAD, D), jnp.float32)] * 4,
    scratch_types=[
        pltpu.VMEM((EPW, EC), jnp.int32),
        pltpu.VMEM((EC, D), jnp.float32),
        pltpu.VMEM((EC, D), jnp.float32),
        pltpu.VMEM((EC, D), jnp.float32),
        pltpu.VMEM((EC, D), jnp.float32),
        pltpu.VMEM((EC, D), jnp.float32),
        pltpu.SemaphoreType.DMA,
        pltpu.SemaphoreType.DMA,
    ],
)


MG = 15
MBPW = 1200 // (8 * MG)
MB_C0 = 13


def _mean_body(y, adjf, m, idx_v, r0, r1, r2, out_v, sem):
    c = lax.axis_index("c")
    s = lax.axis_index("s")
    start = s * (2 * MBPW) + jnp.where(c == 0, 0, MB_C0)
    cnt = jnp.where(c == 0, MB_C0, 2 * MBPW - MB_C0)
    rv = (r0, r1, r2)

    def block(b, carry):
        blk = start + b
        gbase = blk * MG
        pltpu.sync_copy(adjf.at[blk], idx_v)
        cps = {
            0: pltpu.async_copy(y.at[idx_v.at[0]], r0, sem),
            1: pltpu.async_copy(y.at[idx_v.at[1]], r1, sem),
        }
        for gi in range(MG):
            if gi + 2 < MG:
                cps[(gi + 2) % 3] = pltpu.async_copy(
                    y.at[idx_v.at[gi + 2]], rv[(gi + 2) % 3], sem
                )
            cps[gi % 3].wait()
            rows = rv[gi % 3]

            def node(n, c2):
                rid = (gbase + gi) * 8 + n
                scale = jnp.where(rid < R, 1.0 / DEG, 0.0)
                for c in range(D // 16):
                    vs = [
                        rows[n * DEG + j, pl.ds(c * 16, 16)]
                        for j in range(DEG)
                    ]
                    while len(vs) > 1:
                        vs = [vs[i] + vs[i + 1] for i in range(0, len(vs), 2)]
                    out_v[gi * 8 + n, pl.ds(c * 16, 16)] = vs[0] * scale
                return c2

            lax.fori_loop(0, 8, node, 0)
        pltpu.sync_copy(out_v, m.at[pl.ds(gbase * 8, 8 * MG)])
        return carry

    lax.fori_loop(0, cnt, block, 0)


_mean_call = pl.kernel(
    _mean_body,
    mesh=_mesh,
    out_type=jax.ShapeDtypeStruct((RPAD, D), jnp.float32),
    scratch_types=[
        pltpu.VMEM((MG, 128), jnp.int32),
        pltpu.VMEM((128, D), jnp.float32),
        pltpu.VMEM((128, D), jnp.float32),
        pltpu.VMEM((128, D), jnp.float32),
        pltpu.VMEM((8 * MG, D), jnp.float32),
        pltpu.SemaphoreType.DMA,
    ],
)


def _hop_body(spf, mpf, spb, mpb, wsf, wnf, wsb, wnb, osf, oyf, osb, oyb):
    rid = lax.broadcasted_iota(jnp.int32, (TILE, 1), 0) + pl.program_id(0) * TILE
    mask = rid < R
    for sp, mp, ws, wn, os_, oy in (
        (spf, mpf, wsf, wnf, osf, oyf),
        (spb, mpb, wsb, wnb, osb, oyb),
    ):
        a = jnp.maximum(sp[...], 0.0)
        bb = jnp.maximum(mp[...], 0.0)
        w = ws[...]
        s_out = jnp.dot(a, w[:D], preferred_element_type=jnp.float32) + jnp.dot(
            bb, w[D:], preferred_element_type=jnp.float32
        )
        os_[...] = jnp.where(mask, s_out, 0.0)
        w = wn[...]
        y_out = jnp.dot(a, w[:D], preferred_element_type=jnp.float32) + jnp.dot(
            bb, w[D:], preferred_element_type=jnp.float32
        )
        oy[...] = jnp.where(mask, y_out, 0.0)


_hop_call = pl.pallas_call(
    _hop_body,
    grid=(RPAD // TILE,),
    in_specs=[pl.BlockSpec((TILE, D), lambda i: (i, 0))] * 4
    + [pl.BlockSpec((2 * D, D), lambda i: (0, 0))] * 4,
    out_specs=[pl.BlockSpec((TILE, D), lambda i: (i, 0))] * 4,
    out_shape=[jax.ShapeDtypeStruct((RPAD, D), jnp.float32)] * 4,
)


FB = 288


def _final_body(p0, p1, p2, p3, meta, hid, pool, meta_v, b0, b1,
                pool_v, sem, osem):
    w = _wid()
    pltpu.sync_copy(meta, meta_v)
    parts = (p0, p1, p2, p3)
    bufs = (b0, b1)
    units = []
    for gi in range(GPW):
        g = jnp.minimum(w + NW * gi, B - 1)
        for k in range(4):
            units.append((g, k))
    U = len(units)

    def start_in(u):
        g, k = units[u]
        s = meta_v[g, :][0]
        s8 = pl.multiple_of(s & ~7, 8)
        return pltpu.async_copy(parts[k].at[pl.ds(s8, FB)], bufs[u % 2], sem)

    in_cp = {0: start_in(0), 1: start_in(1)}
    out_cp = {}

    for u in range(U):
        g, k = units[u]
        buf = bufs[u % 2]
        mrow = meta_v[g, :]
        s = mrow[0]
        n = mrow[1]
        off = s - (s & ~7)
        in_cp[u % 2].wait()

        def row(r, cc):
            pred = r < n
            out = []
            for v in range(D // 16):
                x = jnp.maximum(buf[r + off, pl.ds(v * 16, 16)], 0.0)
                x = jnp.where(pred, x, 0.0)
                buf[r + off, pl.ds(v * 16, 16)] = x
                out.append(jnp.maximum(cc[v], x))
            return tuple(out)

        car = tuple(jnp.zeros((16,), jnp.float32) for _ in range(D // 16))
        car = lax.fori_loop(0, L, row, car)
        for v in range(D // 16):
            pool_v[pl.ds(k * D + v * 16, 16)] = car[v]
        out_cp[u % 2] = pltpu.async_copy(
            buf.at[pl.ds(off, L)], hid.at[g, :, pl.ds(k * D, D)], osem
        )
        if u + 2 < U:
            out_cp[u % 2].wait()
            in_cp[u % 2] = start_in(u + 2)
        if k == 3:
            pltpu.sync_copy(pool_v, pool.at[g])
    for cp in out_cp.values():
        cp.wait()


_final_call = pl.kernel(
    _final_body,
    mesh=_mesh,
    out_type=[
        jax.ShapeDtypeStruct((B, L, 4 * D), jnp.float32),
        jax.ShapeDtypeStruct((B, 4 * D), jnp.float32),
    ],
    scratch_types=[
        pltpu.VMEM((288, 16), jnp.int32),
        pltpu.VMEM((FB, D), jnp.float32),
        pltpu.VMEM((FB, D), jnp.float32),
        pltpu.VMEM((4 * D,), jnp.float32),
        pltpu.SemaphoreType.DMA,
        pltpu.SemaphoreType.DMA,
    ],
)


def _chunk_adj(adj):
    a = jnp.zeros((RPAD, DEG), jnp.int32).at[:R].set(adj.astype(jnp.int32))
    return a.reshape(RPAD * DEG // (128 * MG), MG, 128)


def kernel(fw_adjs, bw_adjs, features, num_nodes, emb, Wsf0, Wnf0, Wsf, Wnf,
           Wsb0, Wnb0, Wsb, Wnb):
    featp = jnp.zeros((RPAD,), jnp.int32).at[:R].set(features.astype(jnp.int32))
    featp = featp.reshape(NW, EPW, EC)
    fadj = _chunk_adj(fw_adjs)
    badj = _chunk_adj(bw_adjs)
    embp = jnp.zeros((VPAD, D), jnp.float32).at[:VOCAB1].set(emb)

    tsf, tyf, tsb, tyb = _tables_call(embp, Wsf0, Wnf0, Wsb0, Wnb0)
    sf, yf, sb, yb = _embed_call(featp, tsf, tyf, tsb, tyb)
    mf = _mean_call(yf, fadj)
    mb = _mean_call(yb, badj)
    for h in range(1, HOPS):
        sf, yf, sb, yb = _hop_call(sf, mf, sb, mb, Wsf[h - 1], Wnf[h - 1],
                                   Wsb[h - 1], Wnb[h - 1])
        mf = _mean_call(yf, fadj)
        mb = _mean_call(yb, badj)

    nn = num_nodes.astype(jnp.int32)
    csum = jnp.cumsum(nn)
    starts = csum - nn
    meta = jnp.zeros((288, 16), jnp.int32)
    meta = meta.at[:B, 0].set(starts).at[:B, 1].set(nn)

    hidden, pooled = _final_call(sf, mf, sb, mb, meta)
    return hidden, pooled, pooled
